# Initial kernel scaffold; baseline (speedup 1.0000x reference)
import jax
import jax.numpy as jnp
from jax.experimental import pallas as pl


def kernel(node_weight, edge_index, edge_weight, W1, We1, al1, ar1, ae1, b1, W2, We2, al2, ar2, ae2, b2, W3, We3, al3, ar3, ae3, b3, gamma, beta):
    raise NotImplementedError("write your pallas kernel here")


# speedup vs baseline: 5.7050x; 5.7050x over previous
import dataclasses
import functools

import jax
import jax.numpy as jnp
from jax import lax
from jax.experimental import pallas as pl
from jax.experimental.pallas import tpu as pltpu
from jax.experimental.pallas import tpu_sc as plsc

NN = 10000
EE = 160000
DD = 256

EP = 163840
NP = 10240
EPT1 = EP // 32
EPT3 = EP // 16
NEG = -1.0e30

_VMESH = plsc.VectorSubcoreMesh(core_axis_name="core", subcore_axis_name="subcore")


def _sc_params():
    cp = pltpu.CompilerParams()
    if "needs_layout_passes" in pltpu.CompilerParams.__dataclass_fields__:
        cp = dataclasses.replace(cp, needs_layout_passes=False)
    return cp


def _ee_body(ew_ref, we1, ae1, we2, ae2, we3, ae3, out_ref):
    blk = ew_ref[...]
    for i, (we, ae) in enumerate(((we1, ae1), (we2, ae2), (we3, ae3))):
        v = (we[...] * ae[...]).sum(axis=1)
        out_ref[i, :] = (blk * v[None, :]).sum(axis=1)


def _tc_ee(ew, we1, ae1, we2, ae2, we3, ae3):
    wspec = pl.BlockSpec((DD, DD), lambda i: (0, 0))
    aspec = pl.BlockSpec((1, DD), lambda i: (0, 0))
    return pl.pallas_call(
        _ee_body,
        grid=(125,),
        in_specs=[pl.BlockSpec((1280, DD), lambda i: (i, 0)),
                  wspec, aspec, wspec, aspec, wspec, aspec],
        out_specs=pl.BlockSpec((3, 1280), lambda i: (0, i)),
        out_shape=jax.ShapeDtypeStruct((3, EE), jnp.float32),
    )(ew, we1, ae1.reshape(1, DD), we2, ae2.reshape(1, DD),
      we3, ae3.reshape(1, DD))


def _proj_body(x_ref, w_ref, al_ref, ar_ref, el_ref, er_ref):
    x = x_ref[...]
    wa = (w_ref[...] * al_ref[...]).sum(axis=1)
    wr = (w_ref[...] * ar_ref[...]).sum(axis=1)
    el_ref[...] = (x * wa[None, :]).sum(axis=1, keepdims=True)
    er_ref[...] = (x * wr[None, :]).sum(axis=1, keepdims=True)


def _tc_proj(x, w, al, ar):
    return pl.pallas_call(
        _proj_body,
        in_specs=[pl.BlockSpec((NN, DD), lambda: (0, 0)),
                  pl.BlockSpec((DD, DD), lambda: (0, 0)),
                  pl.BlockSpec((1, DD), lambda: (0, 0)),
                  pl.BlockSpec((1, DD), lambda: (0, 0))],
        out_specs=[pl.BlockSpec((NN, 1), lambda: (0, 0)),
                   pl.BlockSpec((NN, 1), lambda: (0, 0))],
        out_shape=[jax.ShapeDtypeStruct((NN, 1), jnp.float32),
                   jax.ShapeDtypeStruct((NN, 1), jnp.float32)],
    )(x, w, al.reshape(1, DD), ar.reshape(1, DD))


def _mm_body(ax0, ax1, aeh0, aeh1, w_ref, we_ref, b_ref, h_ref):
    w = w_ref[...]
    we = we_ref[...]
    acc = jnp.dot(ax0[...], w[0:128, :], preferred_element_type=jnp.float32)
    acc += jnp.dot(ax1[...], w[128:256, :], preferred_element_type=jnp.float32)
    acc += jnp.dot(aeh0[...], we[0:128, :], preferred_element_type=jnp.float32)
    acc += jnp.dot(aeh1[...], we[128:256, :], preferred_element_type=jnp.float32)
    h_ref[...] = acc + b_ref[...]


def _tc_mm(ax0, ax1, aeh0, aeh1, w, we, b):
    hspec = pl.BlockSpec((2000, 128), lambda i: (i, 0))
    wspec = pl.BlockSpec((DD, DD), lambda i: (0, 0))
    return pl.pallas_call(
        _mm_body,
        grid=(5,),
        in_specs=[hspec, hspec, hspec, hspec, wspec, wspec,
                  pl.BlockSpec((1, DD), lambda i: (0, 0))],
        out_specs=pl.BlockSpec((2000, DD), lambda i: (i, 0)),
        out_shape=jax.ShapeDtypeStruct((NN, DD), jnp.float32),
    )(ax0, ax1, aeh0, aeh1, w, we, b.reshape(1, DD))


def _bn_body(with_next, h_ref, g_ref, bt_ref, wn_ref, aln_ref, arn_ref,
             out_ref, el_ref, er_ref):
    h = h_ref[...]
    mu = jnp.mean(h, axis=0)
    xc = h - mu[None, :]
    var = jnp.mean(xc * xc, axis=0)
    y = g_ref[...] * xc * lax.rsqrt(var + 1e-5)[None, :] + bt_ref[...]
    if with_next:
        y = jnp.where(y > 0, y, jnp.exp(y) - 1.0)
        wa = (wn_ref[...] * aln_ref[...]).sum(axis=1)
        wr = (wn_ref[...] * arn_ref[...]).sum(axis=1)
        el_ref[...] = (y * wa[None, :]).sum(axis=1, keepdims=True)
        er_ref[...] = (y * wr[None, :]).sum(axis=1, keepdims=True)
    else:
        el_ref[...] = jnp.zeros_like(el_ref)
        er_ref[...] = jnp.zeros_like(er_ref)
    out_ref[...] = y


def _tc_bn(h, gamma, beta, wn, aln, arn, with_next):
    return pl.pallas_call(
        functools.partial(_bn_body, with_next),
        in_specs=[pl.BlockSpec((NN, DD), lambda: (0, 0)),
                  pl.BlockSpec((1, DD), lambda: (0, 0)),
                  pl.BlockSpec((1, DD), lambda: (0, 0)),
                  pl.BlockSpec((DD, DD), lambda: (0, 0)),
                  pl.BlockSpec((1, DD), lambda: (0, 0)),
                  pl.BlockSpec((1, DD), lambda: (0, 0))],
        out_specs=[pl.BlockSpec((NN, DD), lambda: (0, 0)),
                   pl.BlockSpec((NN, 1), lambda: (0, 0)),
                   pl.BlockSpec((NN, 1), lambda: (0, 0))],
        out_shape=[jax.ShapeDtypeStruct((NN, DD), jnp.float32),
                   jax.ShapeDtypeStruct((NN, 1), jnp.float32),
                   jax.ShapeDtypeStruct((NN, 1), jnp.float32)],
    )(h, gamma.reshape(1, DD), beta.reshape(1, DD), wn,
      aln.reshape(1, DD), arn.reshape(1, DD))


def _inv_body(d_ref, inv_ref):
    inv_ref[...] = 1.0 / jnp.maximum(d_ref[0:1, :] + d_ref[1:2, :], 1e-9)


def _tc_inv(dpart):
    return pl.pallas_call(
        _inv_body,
        in_specs=[pl.BlockSpec((2, NP), lambda: (0, 0))],
        out_specs=pl.BlockSpec((1, NP), lambda: (0, 0)),
        out_shape=jax.ShapeDtypeStruct((1, NP), jnp.float32),
    )(dpart)


def _k1_body(src_h, dst_h, ee_h, el_h, er_h, lg_h, pmax_h,
             el_v, er_v, src_v, dst_v, ee_v, lg_v, mx_v):
    wid = lax.axis_index("core") * 16 + lax.axis_index("subcore")
    e0 = wid * EPT1
    pltpu.sync_copy(el_h, el_v)
    pltpu.sync_copy(er_h, er_v)
    pltpu.sync_copy(src_h.at[pl.ds(e0, EPT1)], src_v)
    pltpu.sync_copy(dst_h.at[pl.ds(e0, EPT1)], dst_v)
    pltpu.sync_copy(ee_h.at[pl.ds(e0, EPT1)], ee_v)

    def body(i, mx):
        sl = pl.ds(i * 16, 16)
        logit = (plsc.load_gather(el_v, [src_v[sl]])
                 + plsc.load_gather(er_v, [dst_v[sl]])
                 + ee_v[sl])
        logit = jnp.maximum(logit, logit * 0.2)
        lg_v[sl] = logit
        return jnp.maximum(mx, logit)

    mx = lax.fori_loop(0, EPT1 // 16, body,
                       jnp.full((16,), -3e38, jnp.float32))
    mx_v[...] = mx
    pltpu.sync_copy(lg_v, lg_h.at[pl.ds(e0, EPT1)])
    pltpu.sync_copy(mx_v, pmax_h.at[wid])


@functools.partial(
    pl.kernel,
    out_type=(jax.ShapeDtypeStruct((EP,), jnp.float32),
              jax.ShapeDtypeStruct((32, 16), jnp.float32)),
    mesh=_VMESH,
    compiler_params=_sc_params(),
    scratch_types=[pltpu.VMEM((NN,), jnp.float32),
                   pltpu.VMEM((NN,), jnp.float32),
                   pltpu.VMEM((EPT1,), jnp.int32),
                   pltpu.VMEM((EPT1,), jnp.int32),
                   pltpu.VMEM((EPT1,), jnp.float32),
                   pltpu.VMEM((EPT1,), jnp.float32),
                   pltpu.VMEM((16,), jnp.float32)],
)
def _sc_logits(*args):
    _k1_body(*args)


def _k2_body(lg_h, pmax_h, dst2_h, u_h, dpart_h,
             pm_v, lg_v, dst2_v, zb_v, denom_s):
    core = lax.axis_index("core")
    sub = lax.axis_index("subcore")
    wid = core * 16 + sub
    e0 = wid * EPT1
    n0 = sub * (NP // 16)

    pltpu.sync_copy(pmax_h, pm_v)

    def mbody(i, mx):
        return jnp.maximum(mx, pm_v[i, :])
    mxv = lax.fori_loop(0, 32, mbody, jnp.full((16,), -3e38, jnp.float32))
    m = jnp.max(mxv)

    @pl.loop(0, (NP // 16) // 16)
    def _(i):
        zb_v[pl.ds(i * 16, 16)] = jnp.zeros((16,), jnp.float32)
    pltpu.sync_copy(zb_v, denom_s.at[pl.ds(n0, NP // 16)])
    plsc.subcore_barrier()

    pltpu.sync_copy(lg_h.at[pl.ds(e0, EPT1)], lg_v)
    pltpu.sync_copy(dst2_h.at[pl.ds(wid * (EPT1 // 128), EPT1 // 128)], dst2_v)

    @pl.loop(0, EPT1 // 16)
    def _(i):
        sl = pl.ds(i * 16, 16)
        lg_v[sl] = jnp.exp(lg_v[sl] - m)

    @pl.loop(0, EPT1 // 128)
    def _(j):
        pltpu.sync_copy(lg_v.at[pl.ds(j * 128, 128)],
                        denom_s.at[dst2_v.at[j]], add=True)

    plsc.subcore_barrier()
    pltpu.sync_copy(denom_s.at[pl.ds(n0, NP // 16)],
                    dpart_h.at[core, pl.ds(n0, NP // 16)])
    pltpu.sync_copy(lg_v, u_h.at[pl.ds(e0, EPT1)])


@functools.partial(
    pl.kernel,
    out_type=(jax.ShapeDtypeStruct((EP,), jnp.float32),
              jax.ShapeDtypeStruct((2, NP), jnp.float32)),
    mesh=_VMESH,
    compiler_params=_sc_params(),
    scratch_types=[pltpu.VMEM((32, 16), jnp.float32),
                   pltpu.VMEM((EPT1,), jnp.float32),
                   pltpu.VMEM((EPT1 // 128, 128), jnp.int32),
                   pltpu.VMEM((NP // 16,), jnp.float32),
                   pltpu.VMEM_SHARED((NP,), jnp.float32)],
)
def _sc_denom(*args):
    _k2_body(*args)


_KQ = 2048


def _k3_body(gather, u_h, inv_h, src_h, dst2_h, tab_h, agg_h,
             inv_v, uq_v, sq_v, dq_v, rows_v, acc_s):
    core = lax.axis_index("core")
    sub = lax.axis_index("subcore")
    e0 = sub * EPT3
    n0 = sub * (NP // 16)

    pltpu.sync_copy(inv_h, inv_v)

    @pl.loop(0, 128)
    def _(r):
        for k in range(8):
            rows_v[r, pl.ds(k * 16, 16)] = jnp.zeros((16,), jnp.float32)
    for k in range(NP // 16 // 128):
        pltpu.sync_copy(rows_v, acc_s.at[pl.ds(n0 + k * 128, 128)])
    plsc.subcore_barrier()

    @pl.loop(0, EPT3 // _KQ)
    def _(q):
        eq = e0 + q * _KQ
        pltpu.sync_copy(u_h.at[pl.ds(eq, _KQ)], uq_v)
        pltpu.sync_copy(
            dst2_h.at[pl.ds(sub * (EPT3 // 128) + q * (_KQ // 128),
                            _KQ // 128)], dq_v)
        if gather:
            pltpu.sync_copy(src_h.at[pl.ds(eq, _KQ)], sq_v)

        @pl.loop(0, _KQ // 128)
        def _(r):
            for l in range(8):
                sl = pl.ds(r * 128 + l * 16, 16)
                dd = dq_v[r, pl.ds(l * 16, 16)]
                uq_v[sl] = uq_v[sl] * plsc.load_gather(inv_v, [dd])

        @pl.loop(0, _KQ // 128)
        def _(j):
            if gather:
                pltpu.sync_copy(
                    tab_h.at[core].at[sq_v.at[pl.ds(j * 128, 128)]], rows_v)
            else:
                off = eq + j * 128
                off = jnp.where(off + 128 <= EE, off, 0)
                pltpu.sync_copy(tab_h.at[pl.ds(off, 128), core], rows_v)

            @pl.loop(0, 8)
            def _(g):
                a16 = uq_v[pl.ds(j * 128 + g * 16, 16)]
                for ri in range(16):
                    r = g * 16 + ri
                    a = a16[ri]
                    for k in range(8):
                        sl = pl.ds(k * 16, 16)
                        rows_v[r, sl] = rows_v[r, sl] * a

            pltpu.sync_copy(rows_v, acc_s.at[dq_v.at[j]], add=True)

    plsc.subcore_barrier()
    for k in range(NP // 16 // 128):
        sl = pl.ds(n0 + k * 128, 128)
        pltpu.sync_copy(acc_s.at[sl], agg_h.at[core, sl])


def _make_k3(gather):
    return functools.partial(
        pl.kernel,
        out_type=jax.ShapeDtypeStruct((2, NP, 128), jnp.float32),
        mesh=_VMESH,
        compiler_params=_sc_params(),
        scratch_types=[pltpu.VMEM((NP,), jnp.float32),
                       pltpu.VMEM((_KQ,), jnp.float32),
                       pltpu.VMEM((_KQ,), jnp.int32),
                       pltpu.VMEM((_KQ // 128, 128), jnp.int32),
                       pltpu.VMEM((128, 128), jnp.float32),
                       pltpu.VMEM_SHARED((NP, 128), jnp.float32)],
    )(functools.partial(_k3_body, gather))


_sc_agg_gather = _make_k3(True)
_sc_agg_linear = _make_k3(False)


def kernel(node_weight, edge_index, edge_weight,
           W1, We1, al1, ar1, ae1, b1,
           W2, We2, al2, ar2, ae2, b2,
           W3, We3, al3, ar3, ae3, b3,
           gamma, beta):
    f32 = jnp.float32
    src = edge_index[0]
    dst = edge_index[1]
    pad = jnp.zeros((EP - EE,), jnp.int32)
    src_p = jnp.concatenate([src, pad])
    dst_p = jnp.concatenate([dst, pad])
    dst2 = dst_p.reshape(EP // 128, 128)
    ew3 = edge_weight.reshape(EE, 2, 128)

    ee = _tc_ee(edge_weight, We1, ae1, We2, ae2, We3, ae3)
    ee = jnp.concatenate([ee, jnp.full((3, EP - EE), NEG, f32)], axis=1)

    el, er = _tc_proj(node_weight, W1, al1, ar1)

    layers = ((W1, We1, b1, W2, al2, ar2),
              (W2, We2, b2, W3, al3, ar3),
              (W3, We3, b3, W1, al1, ar1))
    x = node_weight
    for li, (w, we, b, wn, aln, arn) in enumerate(layers):
        xs = x.reshape(NN, 2, 128).transpose(1, 0, 2)
        lg, pmax = _sc_logits(src_p, dst_p, ee[li], el[:, 0], er[:, 0])
        u, dpart = _sc_denom(lg, pmax, dst2)
        inv = _tc_inv(dpart)[0]
        aggx = _sc_agg_gather(u, inv, src_p, dst2, xs)
        agge = _sc_agg_linear(u, inv, src_p, dst2, ew3)
        h = _tc_mm(aggx[0, :NN, :], aggx[1, :NN, :],
                   agge[0, :NN, :], agge[1, :NN, :], w, we, b)
        x, el, er = _tc_bn(h, gamma, beta, wn, aln, arn, with_next=(li < 2))
    return x


# speedup vs baseline: 7.4083x; 1.2986x over previous
import dataclasses
import functools

import jax
import jax.numpy as jnp
from jax import lax
from jax.experimental import pallas as pl
from jax.experimental.pallas import tpu as pltpu
from jax.experimental.pallas import tpu_sc as plsc

NN = 10000
EE = 160000
DD = 256

EP = 163840
NP = 10240
EPT1 = EP // 32
EPT3 = EP // 16
NEG = -1.0e30

_VMESH = plsc.VectorSubcoreMesh(core_axis_name="core", subcore_axis_name="subcore")


def _sc_params():
    cp = pltpu.CompilerParams()
    if "needs_layout_passes" in pltpu.CompilerParams.__dataclass_fields__:
        cp = dataclasses.replace(cp, needs_layout_passes=False)
    return cp


def _ee_body(ew_ref, we1, ae1, we2, ae2, we3, ae3, out_ref):
    blk = ew_ref[...]
    for i, (we, ae) in enumerate(((we1, ae1), (we2, ae2), (we3, ae3))):
        v = (we[...] * ae[...]).sum(axis=1)
        out_ref[i, :] = (blk * v[None, :]).sum(axis=1)


def _tc_ee(ew, we1, ae1, we2, ae2, we3, ae3):
    wspec = pl.BlockSpec((DD, DD), lambda i: (0, 0))
    aspec = pl.BlockSpec((1, DD), lambda i: (0, 0))
    return pl.pallas_call(
        _ee_body,
        grid=(125,),
        in_specs=[pl.BlockSpec((1280, DD), lambda i: (i, 0)),
                  wspec, aspec, wspec, aspec, wspec, aspec],
        out_specs=pl.BlockSpec((3, 1280), lambda i: (0, i)),
        out_shape=jax.ShapeDtypeStruct((3, EE), jnp.float32),
    )(ew, we1, ae1.reshape(1, DD), we2, ae2.reshape(1, DD),
      we3, ae3.reshape(1, DD))


def _proj_body(x_ref, w_ref, al_ref, ar_ref, el_ref, er_ref):
    x = x_ref[...]
    wa = (w_ref[...] * al_ref[...]).sum(axis=1)
    wr = (w_ref[...] * ar_ref[...]).sum(axis=1)
    el_ref[...] = (x * wa[None, :]).sum(axis=1, keepdims=True)
    er_ref[...] = (x * wr[None, :]).sum(axis=1, keepdims=True)


def _tc_proj(x, w, al, ar):
    return pl.pallas_call(
        _proj_body,
        in_specs=[pl.BlockSpec((NN, DD), lambda: (0, 0)),
                  pl.BlockSpec((DD, DD), lambda: (0, 0)),
                  pl.BlockSpec((1, DD), lambda: (0, 0)),
                  pl.BlockSpec((1, DD), lambda: (0, 0))],
        out_specs=[pl.BlockSpec((NN, 1), lambda: (0, 0)),
                   pl.BlockSpec((NN, 1), lambda: (0, 0))],
        out_shape=[jax.ShapeDtypeStruct((NN, 1), jnp.float32),
                   jax.ShapeDtypeStruct((NN, 1), jnp.float32)],
    )(x, w, al.reshape(1, DD), ar.reshape(1, DD))


def _mm_body(ax0, ax1, aeh0, aeh1, w_ref, we_ref, b_ref, h_ref):
    w = w_ref[...]
    we = we_ref[...]
    acc = jnp.dot(ax0[...], w[0:128, :], preferred_element_type=jnp.float32)
    acc += jnp.dot(ax1[...], w[128:256, :], preferred_element_type=jnp.float32)
    acc += jnp.dot(aeh0[...], we[0:128, :], preferred_element_type=jnp.float32)
    acc += jnp.dot(aeh1[...], we[128:256, :], preferred_element_type=jnp.float32)
    h_ref[...] = acc + b_ref[...]


def _tc_mm(ax0, ax1, aeh0, aeh1, w, we, b):
    hspec = pl.BlockSpec((2000, 128), lambda i: (i, 0))
    wspec = pl.BlockSpec((DD, DD), lambda i: (0, 0))
    return pl.pallas_call(
        _mm_body,
        grid=(5,),
        in_specs=[hspec, hspec, hspec, hspec, wspec, wspec,
                  pl.BlockSpec((1, DD), lambda i: (0, 0))],
        out_specs=pl.BlockSpec((2000, DD), lambda i: (i, 0)),
        out_shape=jax.ShapeDtypeStruct((NN, DD), jnp.float32),
    )(ax0, ax1, aeh0, aeh1, w, we, b.reshape(1, DD))


def _bn_body(with_next, h_ref, g_ref, bt_ref, wn_ref, aln_ref, arn_ref,
             out_ref, el_ref, er_ref):
    h = h_ref[...]
    mu = jnp.mean(h, axis=0)
    xc = h - mu[None, :]
    var = jnp.mean(xc * xc, axis=0)
    y = g_ref[...] * xc * lax.rsqrt(var + 1e-5)[None, :] + bt_ref[...]
    if with_next:
        y = jnp.where(y > 0, y, jnp.exp(y) - 1.0)
        wa = (wn_ref[...] * aln_ref[...]).sum(axis=1)
        wr = (wn_ref[...] * arn_ref[...]).sum(axis=1)
        el_ref[...] = (y * wa[None, :]).sum(axis=1, keepdims=True)
        er_ref[...] = (y * wr[None, :]).sum(axis=1, keepdims=True)
    else:
        el_ref[...] = jnp.zeros_like(el_ref)
        er_ref[...] = jnp.zeros_like(er_ref)
    out_ref[...] = y


def _tc_bn(h, gamma, beta, wn, aln, arn, with_next):
    return pl.pallas_call(
        functools.partial(_bn_body, with_next),
        in_specs=[pl.BlockSpec((NN, DD), lambda: (0, 0)),
                  pl.BlockSpec((1, DD), lambda: (0, 0)),
                  pl.BlockSpec((1, DD), lambda: (0, 0)),
                  pl.BlockSpec((DD, DD), lambda: (0, 0)),
                  pl.BlockSpec((1, DD), lambda: (0, 0)),
                  pl.BlockSpec((1, DD), lambda: (0, 0))],
        out_specs=[pl.BlockSpec((NN, DD), lambda: (0, 0)),
                   pl.BlockSpec((NN, 1), lambda: (0, 0)),
                   pl.BlockSpec((NN, 1), lambda: (0, 0))],
        out_shape=[jax.ShapeDtypeStruct((NN, DD), jnp.float32),
                   jax.ShapeDtypeStruct((NN, 1), jnp.float32),
                   jax.ShapeDtypeStruct((NN, 1), jnp.float32)],
    )(h, gamma.reshape(1, DD), beta.reshape(1, DD), wn,
      aln.reshape(1, DD), arn.reshape(1, DD))


def _inv_body(d_ref, inv_ref):
    inv_ref[...] = 1.0 / jnp.maximum(d_ref[0:1, :] + d_ref[1:2, :], 1e-9)


def _tc_inv(dpart):
    return pl.pallas_call(
        _inv_body,
        in_specs=[pl.BlockSpec((2, NP), lambda: (0, 0))],
        out_specs=pl.BlockSpec((1, NP), lambda: (0, 0)),
        out_shape=jax.ShapeDtypeStruct((1, NP), jnp.float32),
    )(dpart)


def _k12_body(src_h, dst_h, ee_h, el_h, er_h, dst2_h, u_h, dpart_h,
              el_v, er_v, src_v, dst_v, ee_v, u_v, dst2_v, zb_v, denom_s):
    core = lax.axis_index("core")
    sub = lax.axis_index("subcore")
    wid = core * 16 + sub
    e0 = wid * EPT1
    n0 = sub * (NP // 16)

    @pl.loop(0, (NP // 16) // 16)
    def _(i):
        zb_v[pl.ds(i * 16, 16)] = jnp.zeros((16,), jnp.float32)
    pltpu.sync_copy(zb_v, denom_s.at[pl.ds(n0, NP // 16)])

    pltpu.sync_copy(el_h, el_v)
    pltpu.sync_copy(er_h, er_v)
    pltpu.sync_copy(src_h.at[pl.ds(e0, EPT1)], src_v)
    pltpu.sync_copy(dst_h.at[pl.ds(e0, EPT1)], dst_v)
    pltpu.sync_copy(ee_h.at[pl.ds(e0, EPT1)], ee_v)
    pltpu.sync_copy(dst2_h.at[pl.ds(wid * (EPT1 // 128), EPT1 // 128)], dst2_v)
    plsc.subcore_barrier()

    @pl.loop(0, EPT1 // 16)
    def _(i):
        sl = pl.ds(i * 16, 16)
        logit = (plsc.load_gather(el_v, [src_v[sl]])
                 + plsc.load_gather(er_v, [dst_v[sl]])
                 + ee_v[sl])
        logit = jnp.maximum(logit, logit * 0.2)
        u_v[sl] = jnp.exp(jnp.minimum(logit, 60.0))

    @pl.loop(0, EPT1 // 128)
    def _(j):
        pltpu.sync_copy(u_v.at[pl.ds(j * 128, 128)],
                        denom_s.at[dst2_v.at[j]], add=True)

    plsc.subcore_barrier()
    pltpu.sync_copy(denom_s.at[pl.ds(n0, NP // 16)],
                    dpart_h.at[core, pl.ds(n0, NP // 16)])
    pltpu.sync_copy(u_v, u_h.at[pl.ds(e0, EPT1)])


@functools.partial(
    pl.kernel,
    out_type=(jax.ShapeDtypeStruct((EP,), jnp.float32),
              jax.ShapeDtypeStruct((2, NP), jnp.float32)),
    mesh=_VMESH,
    compiler_params=_sc_params(),
    scratch_types=[pltpu.VMEM((NN,), jnp.float32),
                   pltpu.VMEM((NN,), jnp.float32),
                   pltpu.VMEM((EPT1,), jnp.int32),
                   pltpu.VMEM((EPT1,), jnp.int32),
                   pltpu.VMEM((EPT1,), jnp.float32),
                   pltpu.VMEM((EPT1,), jnp.float32),
                   pltpu.VMEM((EPT1 // 128, 128), jnp.int32),
                   pltpu.VMEM((NP // 16,), jnp.float32),
                   pltpu.VMEM_SHARED((NP,), jnp.float32)],
)
def _sc_attn(*args):
    _k12_body(*args)


_KQ = 2048


def _scale_rows(rows_b, uq_v, c):
    @pl.loop(0, 8)
    def _(g):
        a16 = uq_v[pl.ds(c * 128 + g * 16, 16)]
        for ri in range(16):
            r = g * 16 + ri
            a = a16[ri]
            for k in range(8):
                sl = pl.ds(k * 16, 16)
                rows_b[r, sl] = rows_b[r, sl] * a


def _k3_body(gather, u_h, inv_h, src_h, dst2_h, tab_h, agg_h,
             inv_v, uq_v, sq_v, dq_v, rows_v, g0, g1, s0, s1, acc_s):
    core = lax.axis_index("core")
    sub = lax.axis_index("subcore")
    e0 = sub * EPT3
    n0 = sub * (NP // 16)

    pltpu.sync_copy(inv_h, inv_v)

    @pl.loop(0, 128)
    def _(r):
        for k in range(8):
            rows_v[0, r, pl.ds(k * 16, 16)] = jnp.zeros((16,), jnp.float32)
    for k in range(NP // 16 // 128):
        pltpu.sync_copy(rows_v.at[0], acc_s.at[pl.ds(n0 + k * 128, 128)])
    plsc.subcore_barrier()

    def wait_read(buf, sem):
        if gather:
            dummy = tab_h.at[core, pl.ds(0, 128)]
        else:
            dummy = tab_h.at[pl.ds(0, 128), core]
        pltpu.make_async_copy(dummy, rows_v.at[buf], sem).wait()

    def wait_scat(buf, sem):
        pltpu.make_async_copy(rows_v.at[buf], acc_s.at[dq_v.at[0]],
                              sem).wait()

    @pl.loop(0, EPT3 // _KQ)
    def _(q):
        eq = e0 + q * _KQ
        pltpu.sync_copy(u_h.at[pl.ds(eq, _KQ)], uq_v)
        pltpu.sync_copy(
            dst2_h.at[pl.ds(sub * (EPT3 // 128) + q * (_KQ // 128),
                            _KQ // 128)], dq_v)
        if gather:
            pltpu.sync_copy(src_h.at[pl.ds(eq, _KQ)], sq_v)

        @pl.loop(0, _KQ // 128)
        def _(r):
            for l in range(8):
                sl = pl.ds(r * 128 + l * 16, 16)
                dd = dq_v[r, pl.ds(l * 16, 16)]
                uq_v[sl] = uq_v[sl] * plsc.load_gather(inv_v, [dd])

        def start_read(c, buf, sem):
            if gather:
                pltpu.async_copy(
                    tab_h.at[core].at[sq_v.at[pl.ds(c * 128, 128)]],
                    rows_v.at[buf], sem)
            else:
                off = eq + c * 128
                off = jnp.where(off + 128 <= EE, off, 0)
                pltpu.async_copy(tab_h.at[pl.ds(off, 128), core],
                                 rows_v.at[buf], sem)

        start_read(0, 0, g0)

        @pl.loop(0, _KQ // 256)
        def _(p):
            c0 = 2 * p
            c1 = 2 * p + 1

            @pl.when(p > 0)
            def _():
                wait_scat(1, s1)
            start_read(c1, 1, g1)
            wait_read(0, g0)
            _scale_rows(rows_v.at[0], uq_v, c0)
            pltpu.async_copy(rows_v.at[0], acc_s.at[dq_v.at[c0]], s0,
                             add=True)

            @pl.when(p + 1 < _KQ // 256)
            def _():
                wait_scat(0, s0)
                start_read(2 * p + 2, 0, g0)
            wait_read(1, g1)
            _scale_rows(rows_v.at[1], uq_v, c1)
            pltpu.async_copy(rows_v.at[1], acc_s.at[dq_v.at[c1]], s1,
                             add=True)

        wait_scat(0, s0)
        wait_scat(1, s1)

    plsc.subcore_barrier()
    for k in range(NP // 16 // 128):
        sl = pl.ds(n0 + k * 128, 128)
        pltpu.sync_copy(acc_s.at[sl], agg_h.at[core, sl])


def _make_k3(gather):
    return functools.partial(
        pl.kernel,
        out_type=jax.ShapeDtypeStruct((2, NP, 128), jnp.float32),
        mesh=_VMESH,
        compiler_params=_sc_params(),
        scratch_types=[pltpu.VMEM((NN,), jnp.float32),
                       pltpu.VMEM((_KQ,), jnp.float32),
                       pltpu.VMEM((_KQ,), jnp.int32),
                       pltpu.VMEM((_KQ // 128, 128), jnp.int32),
                       pltpu.VMEM((2, 128, 128), jnp.float32),
                       pltpu.SemaphoreType.DMA,
                       pltpu.SemaphoreType.DMA,
                       pltpu.SemaphoreType.DMA,
                       pltpu.SemaphoreType.DMA,
                       pltpu.VMEM_SHARED((NP, 128), jnp.float32)],
    )(functools.partial(_k3_body, gather))


_sc_agg_gather = _make_k3(True)
_sc_agg_linear = _make_k3(False)


def kernel(node_weight, edge_index, edge_weight,
           W1, We1, al1, ar1, ae1, b1,
           W2, We2, al2, ar2, ae2, b2,
           W3, We3, al3, ar3, ae3, b3,
           gamma, beta):
    f32 = jnp.float32
    src = edge_index[0]
    dst = edge_index[1]
    pad = jnp.zeros((EP - EE,), jnp.int32)
    src_p = jnp.concatenate([src, pad])
    dst_p = jnp.concatenate([dst, pad])
    dst2 = dst_p.reshape(EP // 128, 128)
    ew3 = edge_weight.reshape(EE, 2, 128)

    ee = _tc_ee(edge_weight, We1, ae1, We2, ae2, We3, ae3)
    ee = jnp.concatenate([ee, jnp.full((3, EP - EE), NEG, f32)], axis=1)

    el, er = _tc_proj(node_weight, W1, al1, ar1)

    layers = ((W1, We1, b1, W2, al2, ar2),
              (W2, We2, b2, W3, al3, ar3),
              (W3, We3, b3, W1, al1, ar1))
    x = node_weight
    for li, (w, we, b, wn, aln, arn) in enumerate(layers):
        xs = x.reshape(NN, 2, 128).transpose(1, 0, 2)
        u, dpart = _sc_attn(src_p, dst_p, ee[li], el[:, 0], er[:, 0], dst2)
        inv = _tc_inv(dpart)[0, :NN]
        aggx = _sc_agg_gather(u, inv, src_p, dst2, xs)
        agge = _sc_agg_linear(u, inv, src_p, dst2, ew3)
        h = _tc_mm(aggx[0, :NN, :], aggx[1, :NN, :],
                   agge[0, :NN, :], agge[1, :NN, :], w, we, b)
        x, el, er = _tc_bn(h, gamma, beta, wn, aln, arn, with_next=(li < 2))
    return x
